# baseline (speedup 1.0000x reference)
import math
from functools import partial

import jax
import jax.numpy as jnp
from jax.experimental import pallas as pl
from jax.experimental.pallas import tpu as pltpu


def _embedder_kernel(t_ref, freqs_ref, w1_ref, b1_ref, w2_ref,
                     b2_ref, o_ref, *, tile_n, sub_rows):
    half = freqs_ref.shape[1]
    freqs = freqs_ref[...]
    b1 = b1_ref[...]
    b2 = b2_ref[...]
    w1c = w1_ref[:half, :]
    w1s = w1_ref[half:, :]
    w2 = w2_ref[...]

    base = pl.program_id(0) * tile_n
    for c in range(tile_n // sub_rows):
        t_sl = t_ref[pl.ds(base + c * sub_rows, sub_rows), :]
        args = t_sl * freqs
        h = (jnp.dot(jnp.cos(args), w1c, preferred_element_type=jnp.float32)
             + jnp.dot(jnp.sin(args), w1s, preferred_element_type=jnp.float32)
             + b1)
        h = h * jax.lax.logistic(h)
        o_ref[pl.ds(c * sub_rows, sub_rows), :] = (
            jnp.dot(h, w2, preferred_element_type=jnp.float32) + b2)


def kernel(t, w1, b1, w2, b2, *, frequency_embedding_size=256,
           max_period=10000, max_tile_n=1024, sub_rows=128):
    N = t.shape[0]
    F = frequency_embedding_size
    half = F // 2
    H = w1.shape[1]
    assert F % 2 == 0, "frequency_embedding_size must be even"
    assert w1.shape[0] == F and w2.shape == (H, H)

    freqs = jnp.exp(
        -math.log(max_period) * jnp.arange(half, dtype=jnp.float32) / half
    ).reshape(1, half)

    tn = min(max_tile_n, -(-N // 8) * 8)
    sub = sub_rows if tn % sub_rows == 0 else tn
    n_pad = -(-N // tn) * tn
    if n_pad == N:
        t_col = t.astype(jnp.float32).reshape(N, 1)
    else:
        t_col = jnp.zeros((n_pad, 1), jnp.float32).at[:N, 0].set(
            t.astype(jnp.float32))

    out = pl.pallas_call(
        partial(_embedder_kernel, tile_n=tn, sub_rows=sub),
        grid=(n_pad // tn,),
        in_specs=[
            pl.BlockSpec((n_pad, 1), lambda i: (0, 0)),
            pl.BlockSpec((1, half), lambda i: (0, 0)),
            pl.BlockSpec((F, H), lambda i: (0, 0)),
            pl.BlockSpec((1, H), lambda i: (0, 0)),
            pl.BlockSpec((H, H), lambda i: (0, 0)),
            pl.BlockSpec((1, H), lambda i: (0, 0)),
        ],
        out_specs=pl.BlockSpec((tn, H), lambda i: (i, 0)),
        out_shape=jax.ShapeDtypeStruct((n_pad, H), jnp.float32),
        compiler_params=pltpu.CompilerParams(
            dimension_semantics=("arbitrary",)),
    )(t_col, freqs, w1, b1.reshape(1, H), w2, b2.reshape(1, H))
    return out[:N]
